# Initial kernel scaffold; baseline (speedup 1.0000x reference)
import jax
import jax.numpy as jnp
from jax.experimental import pallas as pl


def kernel(boxes, scores, overlap_threshold, n_objects_max, topk_only):
    raise NotImplementedError("write your pallas kernel here")


# speedup vs baseline: 9.5376x; 9.5376x over previous
import functools

import jax
import jax.numpy as jnp
from jax import lax
from jax.experimental import pallas as pl
from jax.experimental.pallas import tpu as pltpu
from jax.experimental.pallas import tpu_sc as plsc

_N = 5000
_K = 200
_ROWS = 40
_NPAD = _ROWS * 128
_KROWS = 2
_KPAD = _KROWS * 128
_D = 8


def _nms_body(bx_ref, by_ref, bw_ref, bh_ref, sc_ref, thr_ref, nmax_ref,
              chosen_ref):
    bx = bx_ref[...]
    by = by_ref[...]
    bw = bw_ref[...]
    bh = bh_ref[...]
    scores = sc_ref[...]
    thr = thr_ref[0]
    nmax = nmax_ref[0]

    x1 = bx - 0.5 * bw
    x3 = bx + 0.5 * bw
    y1 = by - 0.5 * bh
    y3 = by + 0.5 * bh
    area = bw * bh

    row = lax.broadcasted_iota(jnp.int32, (_ROWS, 128), 0)
    col = lax.broadcasted_iota(jnp.int32, (_ROWS, 128), 1)
    flat = row * 128 + col
    krow = lax.broadcasted_iota(jnp.int32, (_KROWS, 128), 0)
    kcol = lax.broadcasted_iota(jnp.int32, (_KROWS, 128), 1)
    kflat = krow * 128 + kcol

    def body(k, carry):
        possible, chosen = carry
        masked = scores * possible - (1.0 - possible) * 1e9
        m = jnp.max(masked)
        idx = jnp.min(jnp.where(masked == m, flat, jnp.int32(2**30)))
        sel = flat == idx
        zero = jnp.float32(0.0)
        cbx = jnp.sum(jnp.where(sel, bx, zero))
        cby = jnp.sum(jnp.where(sel, by, zero))
        cbw = jnp.sum(jnp.where(sel, bw, zero))
        cbh = jnp.sum(jnp.where(sel, bh, zero))
        cx1 = cbx - 0.5 * cbw
        cx3 = cbx + 0.5 * cbw
        cy1 = cby - 0.5 * cbh
        cy3 = cby + 0.5 * cbh
        carea = cbw * cbh
        ix = jnp.maximum(jnp.minimum(x3, cx3) - jnp.maximum(x1, cx1), 0.0)
        iy = jnp.maximum(jnp.minimum(y3, cy3) - jnp.maximum(y1, cy1), 0.0)
        inter = ix * iy
        union = area + carea - inter
        iou = inter / jnp.maximum(union, 1e-8)
        keep = jnp.where((iou > thr) | (flat == idx), 0.0, possible)
        new_chosen = jnp.where(kflat == k, idx, chosen)
        active = k < nmax
        possible = jnp.where(active, keep, possible)
        chosen = jnp.where(active, new_chosen, chosen)
        return possible, chosen

    possible0 = (flat < _N).astype(jnp.float32)
    chosen0 = jnp.zeros((_KROWS, 128), jnp.int32)
    _, chosen = lax.fori_loop(0, _K, body, (possible0, chosen0))
    chosen_ref[...] = chosen


_nms_call = pl.pallas_call(
    _nms_body,
    out_shape=jax.ShapeDtypeStruct((_KROWS, 128), jnp.int32),
    in_specs=[
        pl.BlockSpec(memory_space=pltpu.VMEM),
        pl.BlockSpec(memory_space=pltpu.VMEM),
        pl.BlockSpec(memory_space=pltpu.VMEM),
        pl.BlockSpec(memory_space=pltpu.VMEM),
        pl.BlockSpec(memory_space=pltpu.VMEM),
        pl.BlockSpec(memory_space=pltpu.SMEM),
        pl.BlockSpec(memory_space=pltpu.SMEM),
    ],
    out_specs=pl.BlockSpec(memory_space=pltpu.VMEM),
)


@functools.cache
def _make_sc_gather():
    info = plsc.get_sparse_core_info()
    nc, ns = info.num_cores, info.num_subcores
    nw = nc * ns
    b_per_w = _KPAD // nw
    mesh = plsc.VectorSubcoreMesh(core_axis_name="c", subcore_axis_name="s")

    @functools.partial(
        pl.kernel,
        mesh=mesh,
        compiler_params=pltpu.CompilerParams(use_tc_tiling_on_sc=False),
        out_type=jax.ShapeDtypeStruct((_KPAD, _D), jnp.float32),
        scratch_types=[
            pltpu.VMEM((b_per_w,), jnp.int32),
            pltpu.VMEM((b_per_w, _D), jnp.float32),
            pltpu.SemaphoreType.DMA,
        ],
    )
    def gather(table_hbm, idx_hbm, out_hbm, idx_v, rows_v, sem):
        wid = lax.axis_index("s") * nc + lax.axis_index("c")
        base = wid * b_per_w
        pltpu.sync_copy(idx_hbm.at[pl.ds(base, b_per_w)], idx_v)
        pltpu.async_copy(table_hbm.at[idx_v], rows_v, sem).wait()
        pltpu.sync_copy(rows_v, out_hbm.at[pl.ds(base, b_per_w)])

    return gather


def kernel(boxes, scores, overlap_threshold, n_objects_max, topk_only):
    thr = jnp.where(topk_only, jnp.float32(2.0),
                    jnp.asarray(overlap_threshold, jnp.float32))
    nmax = jnp.where(topk_only, jnp.int32(_K),
                     jnp.asarray(n_objects_max, jnp.int32))

    boxes_p = jnp.pad(boxes, ((0, _NPAD - _N), (0, 0)))
    fields = boxes_p.T.reshape(4, _ROWS, 128)
    scores_p = jnp.pad(scores, (0, _NPAD - _N)).reshape(_ROWS, 128)

    chosen2d = _nms_call(fields[0], fields[1], fields[2], fields[3], scores_p,
                         thr.reshape(1), nmax.reshape(1))
    chosen_flat = chosen2d.reshape(_KPAD)

    table = jnp.pad(
        jnp.concatenate([scores[:, None], boxes], axis=1),
        ((0, 0), (0, _D - 5)))
    rows = _make_sc_gather()(table, chosen_flat)

    out = rows[:_K, :5]
    chosen = chosen_flat[:_K]
    return out, chosen
